# baseline (speedup 1.0000x reference)
import functools

import jax
import jax.numpy as jnp
from jax import lax
from jax.experimental import pallas as pl
from jax.experimental.pallas import tpu as pltpu
from jax.experimental.pallas import tpu_sc as plsc

TOP_K = 8


def _mean_body(x_ref, o_ref):
    s = x_ref.shape[1]
    o_ref[...] = (jnp.sum(x_ref[0], axis=0, keepdims=True) * (1.0 / s))[None]


def _l2n(x):
    sq = jnp.sum(x * x, axis=-1, keepdims=True)
    return x * lax.rsqrt(jnp.maximum(sq, 1e-12))


def _simtopk_body(xm_ref, pk_ref, sim_ref, idx_ref, keyn_ref, rs_ref):
    b = xm_ref.shape[0]
    p = pk_ref.shape[0]
    key_norm = _l2n(pk_ref[...])
    x_norm = _l2n(xm_ref[...])
    keyn_ref[...] = key_norm
    sim = lax.dot_general(
        x_norm, key_norm,
        dimension_numbers=(((1,), (1,)), ((), ())),
        preferred_element_type=jnp.float32,
    )
    sim_ref[...] = sim
    ids = lax.broadcasted_iota(jnp.int32, (b, p), 1)
    cur = sim
    total = jnp.zeros((b, 1), jnp.float32)
    for k in range(TOP_K):
        m = jnp.max(cur, axis=1, keepdims=True)
        cand = jnp.where(cur == m, ids, jnp.int32(2**30))
        j = jnp.min(cand, axis=1, keepdims=True)
        idx_ref[:, k:k + 1] = j
        total = total + m
        cur = jnp.where(ids == j, -jnp.inf, cur)
    rs_ref[...] = jnp.sum(total, axis=0, keepdims=True) * (1.0 / b)


def _sc_gather(l, p, length, d, b, k):
    nrow = l * b * k
    nkey = b * k
    nw = 32
    rpw = nrow // nw
    kpw = nkey // nw
    cw = 8
    nchunk = rpw // cw
    row_w = length * d
    mesh = plsc.VectorSubcoreMesh(core_axis_name="c", subcore_axis_name="s")

    @functools.partial(
        pl.kernel,
        mesh=mesh,
        out_type=[
            jax.ShapeDtypeStruct((nrow, row_w), jnp.float32),
            jax.ShapeDtypeStruct((nkey, d), jnp.float32),
        ],
        scratch_types=[
            pltpu.VMEM((rpw,), jnp.int32),
            pltpu.VMEM((kpw,), jnp.int32),
            pltpu.VMEM((cw, row_w), jnp.float32),
            pltpu.VMEM((cw, row_w), jnp.float32),
            pltpu.VMEM((kpw, d), jnp.float32),
            pltpu.SemaphoreType.DMA,
            pltpu.SemaphoreType.DMA,
            pltpu.SemaphoreType.DMA,
            pltpu.SemaphoreType.DMA,
        ],
    )
    def gather_kernel(table_hbm, keyn_hbm, gidx_hbm, kidx_hbm,
                      out1_hbm, out2_hbm,
                      gidx_v, kidx_v, buf0, buf1, krows_v,
                      sg0, sg1, sw, sk):
        wid = lax.axis_index("s") * 2 + lax.axis_index("c")
        base = wid * rpw
        bufs = (buf0, buf1)
        gsems = (sg0, sg1)

        pltpu.sync_copy(gidx_hbm.at[pl.ds(base, rpw)], gidx_v)

        pltpu.sync_copy(kidx_hbm.at[pl.ds(wid * kpw, kpw)], kidx_v)
        hk = pltpu.async_copy(keyn_hbm.at[kidx_v], krows_v, sk)

        def g_start(c):
            return pltpu.async_copy(
                table_hbm.at[gidx_v.at[pl.ds(c * cw, cw)]],
                bufs[c % 2], gsems[c % 2])

        hg = {0: g_start(0)}
        if nchunk > 1:
            hg[1] = g_start(1)
        for c in range(nchunk):
            hg[c].wait()
            hw = pltpu.async_copy(
                bufs[c % 2], out1_hbm.at[pl.ds(base + c * cw, cw)], sw)
            if c + 2 < nchunk:
                hw.wait()
                hg[c + 2] = g_start(c + 2)
            else:
                hw.wait()

        hk.wait()
        pltpu.sync_copy(krows_v, out2_hbm.at[pl.ds(wid * kpw, kpw)])

    return gather_kernel


def kernel(x_embed, prompt, prompt_key):
    b, s, d = x_embed.shape
    l, p, length, d2 = prompt.shape
    k = TOP_K

    x_mean = pl.pallas_call(
        _mean_body,
        grid=(b,),
        in_specs=[pl.BlockSpec((1, s, d), lambda i: (i, 0, 0))],
        out_specs=pl.BlockSpec((1, 1, d), lambda i: (i, 0, 0)),
        out_shape=jax.ShapeDtypeStruct((b, 1, d), jnp.float32),
    )(x_embed)
    x_mean = x_mean.reshape(b, d)

    sim, idx, key_norm, rs = pl.pallas_call(
        _simtopk_body,
        out_shape=[
            jax.ShapeDtypeStruct((b, p), jnp.float32),
            jax.ShapeDtypeStruct((b, k), jnp.int32),
            jax.ShapeDtypeStruct((p, d), jnp.float32),
            jax.ShapeDtypeStruct((1, 1), jnp.float32),
        ],
    )(x_mean, prompt_key)

    flat = idx.reshape(-1)
    offs = (jnp.arange(l, dtype=jnp.int32) * p)[:, None]
    gidx = (flat[None, :] + offs).reshape(l * b * k)
    kidx = flat

    table = prompt.reshape(l * p, length * d)
    out1, out2 = _sc_gather(l, p, length, d, b, k)(table, key_norm, gidx, kidx)

    batched_prompt = out1.reshape(l, b, k * length, d)
    batched_key_norm = out2.reshape(b, k, d)
    reduce_sim = rs.reshape(())
    return (sim, idx, batched_prompt, batched_key_norm, reduce_sim)


# speedup vs baseline: 1.0686x; 1.0686x over previous
import dataclasses
import functools

import jax
import jax.numpy as jnp
from jax import lax
from jax.experimental import pallas as pl
from jax.experimental.pallas import tpu as pltpu
from jax.experimental.pallas import tpu_sc as plsc

TOP_K = 8


def _mean_body(x_ref, o_ref):
    s = x_ref.shape[1]
    o_ref[...] = (jnp.sum(x_ref[0], axis=0, keepdims=True) * (1.0 / s))[None]


def _l2n(x):
    sq = jnp.sum(x * x, axis=-1, keepdims=True)
    return x * lax.rsqrt(jnp.maximum(sq, 1e-12))


def _simtopk_body(xm_ref, pk_ref, sim_ref, idx_ref, keyn_ref, rs_ref):
    b = xm_ref.shape[0]
    p = pk_ref.shape[0]
    key_norm = _l2n(pk_ref[...])
    x_norm = _l2n(xm_ref[...])
    keyn_ref[...] = key_norm
    sim = lax.dot_general(
        x_norm, key_norm,
        dimension_numbers=(((1,), (1,)), ((), ())),
        preferred_element_type=jnp.float32,
    )
    sim_ref[...] = sim
    ids = lax.broadcasted_iota(jnp.int32, (b, p), 1)
    cur = sim
    total = jnp.zeros((b, 1), jnp.float32)
    for k in range(TOP_K):
        m = jnp.max(cur, axis=1, keepdims=True)
        cand = jnp.where(cur == m, ids, jnp.int32(2**30))
        j = jnp.min(cand, axis=1, keepdims=True)
        idx_ref[:, k:k + 1] = j
        total = total + m
        cur = jnp.where(ids == j, -jnp.inf, cur)
    rs_ref[...] = jnp.sum(total, axis=0, keepdims=True) * (1.0 / b)


def _relayout_body(x_ref, o_ref):
    g, length, d = x_ref.shape
    o_ref[...] = x_ref[...].reshape(g * length, d)


def _sc_gather(l, p, length, d, b, k):
    ngrp = l * b
    nkey = b * k
    nw = 32
    gpw = ngrp // nw
    kpw = nkey // nw
    mesh = plsc.VectorSubcoreMesh(core_axis_name="c", subcore_axis_name="s")
    cp = pltpu.CompilerParams()
    if "needs_layout_passes" in pltpu.CompilerParams.__dataclass_fields__:
        cp = dataclasses.replace(cp, needs_layout_passes=False)

    @functools.partial(
        pl.kernel,
        mesh=mesh,
        compiler_params=cp,
        out_type=[
            jax.ShapeDtypeStruct((l * b * k, length, d), jnp.float32),
            jax.ShapeDtypeStruct((nkey, d), jnp.float32),
        ],
        scratch_types=[
            pltpu.VMEM((nkey,), jnp.int32),
            pltpu.VMEM((k, length, d), jnp.float32),
            pltpu.VMEM((k, length, d), jnp.float32),
            pltpu.VMEM((kpw, d), jnp.float32),
            pltpu.SemaphoreType.DMA,
            pltpu.SemaphoreType.DMA,
            pltpu.SemaphoreType.DMA,
            pltpu.SemaphoreType.DMA,
        ],
    )
    def gather_kernel(prompt_hbm, keyn_hbm, idx_hbm,
                      out1_hbm, out2_hbm,
                      idx_v, buf0, buf1, krows_v,
                      sg0, sg1, sw, sk):
        wid = lax.axis_index("s") * 2 + lax.axis_index("c")
        bufs = (buf0, buf1)
        gsems = (sg0, sg1)
        lanes = lax.iota(jnp.int32, 16)

        pltpu.sync_copy(idx_hbm, idx_v)

        hk = pltpu.async_copy(
            keyn_hbm.at[idx_v.at[pl.ds(wid * kpw, kpw)]], krows_v, sk)

        def g_start(j):
            g = wid * gpw + j
            li = g // b
            bi = g % b
            vec = idx_v[pl.ds((bi // 2) * 16, 16)]
            hs = []
            for r in range(k):
                e = (bi % 2) * k + r
                pv = jnp.max(jnp.where(lanes == e, vec, jnp.int32(-2**31)),
                             axis=0)
                hs.append(pltpu.async_copy(
                    prompt_hbm.at[li].at[pv],
                    bufs[j % 2].at[r], gsems[j % 2]))
            return hs

        hg = {0: g_start(0)}
        if gpw > 1:
            hg[1] = g_start(1)
        for j in range(gpw):
            g = wid * gpw + j
            for h in hg.pop(j):
                h.wait()
            hw = pltpu.async_copy(
                bufs[j % 2], out1_hbm.at[pl.ds(g * k, k)], sw)
            hw.wait()
            if j + 2 < gpw:
                hg[j + 2] = g_start(j + 2)

        hk.wait()
        pltpu.sync_copy(krows_v, out2_hbm.at[pl.ds(wid * kpw, kpw)])

    return gather_kernel


def kernel(x_embed, prompt, prompt_key):
    b, s, d = x_embed.shape
    l, p, length, d2 = prompt.shape
    k = TOP_K

    x_mean = pl.pallas_call(
        _mean_body,
        grid=(b,),
        in_specs=[pl.BlockSpec((1, s, d), lambda i: (i, 0, 0))],
        out_specs=pl.BlockSpec((1, 1, d), lambda i: (i, 0, 0)),
        out_shape=jax.ShapeDtypeStruct((b, 1, d), jnp.float32),
    )(x_embed)
    x_mean = x_mean.reshape(b, d)

    sim, idx, key_norm, rs = pl.pallas_call(
        _simtopk_body,
        out_shape=[
            jax.ShapeDtypeStruct((b, p), jnp.float32),
            jax.ShapeDtypeStruct((b, k), jnp.int32),
            jax.ShapeDtypeStruct((p, d), jnp.float32),
            jax.ShapeDtypeStruct((1, 1), jnp.float32),
        ],
    )(x_mean, prompt_key)

    flat = idx.reshape(-1)
    out1p, out2 = _sc_gather(l, p, length, d, b, k)(prompt, key_norm, flat)

    nrow = l * b * k
    gblk = nrow // l
    out1 = pl.pallas_call(
        _relayout_body,
        grid=(l,),
        in_specs=[pl.BlockSpec((gblk, length, d), lambda i: (i, 0, 0))],
        out_specs=pl.BlockSpec((gblk * length, d), lambda i: (i, 0)),
        out_shape=jax.ShapeDtypeStruct((nrow * length, d), jnp.float32),
    )(out1p)

    batched_prompt = out1.reshape(l, b, k * length, d)
    batched_key_norm = out2.reshape(b, k, d)
    reduce_sim = rs.reshape(())
    return (sim, idx, batched_prompt, batched_key_norm, reduce_sim)


# speedup vs baseline: 3.4526x; 3.2311x over previous
import dataclasses
import functools

import jax
import jax.numpy as jnp
from jax import lax
from jax.experimental import pallas as pl
from jax.experimental.pallas import tpu as pltpu
from jax.experimental.pallas import tpu_sc as plsc

TOP_K = 8


def _mean_body(x_ref, o_ref):
    s = x_ref.shape[1]
    o_ref[...] = (jnp.sum(x_ref[0], axis=0, keepdims=True) * (1.0 / s))[None]


def _l2n(x):
    sq = jnp.sum(x * x, axis=-1, keepdims=True)
    return x * lax.rsqrt(jnp.maximum(sq, 1e-12))


def _simtopk_body(xm_ref, pk_ref, sim_ref, idx_ref, keyn_ref, rs_ref):
    b = xm_ref.shape[0]
    p = pk_ref.shape[0]
    key_norm = _l2n(pk_ref[...])
    x_norm = _l2n(xm_ref[...])
    keyn_ref[...] = key_norm
    sim = lax.dot_general(
        x_norm, key_norm,
        dimension_numbers=(((1,), (1,)), ((), ())),
        preferred_element_type=jnp.float32,
    )
    sim_ref[...] = sim
    ids = lax.broadcasted_iota(jnp.int32, (b, p), 1)
    cur = sim
    total = jnp.zeros((b, 1), jnp.float32)
    for k in range(TOP_K):
        m = jnp.max(cur, axis=1, keepdims=True)
        cand = jnp.where(cur == m, ids, jnp.int32(2**30))
        j = jnp.min(cand, axis=1, keepdims=True)
        idx_ref[:, k:k + 1] = j
        total = total + m
        cur = jnp.where(ids == j, -jnp.inf, cur)
    rs_ref[...] = jnp.sum(total, axis=0, keepdims=True) * (1.0 / b)


def _sc_gather(l, p, length, d, b, k):
    nrow = l * b * k * length
    nkey = b * k
    nw = 32
    rpw = nrow // nw
    kpw = nkey // nw
    cw = 48
    nchunk = rpw // cw
    mesh = plsc.VectorSubcoreMesh(core_axis_name="c", subcore_axis_name="s")

    @functools.partial(
        pl.kernel,
        mesh=mesh,
        out_type=[
            jax.ShapeDtypeStruct((nrow, d), jnp.float32),
            jax.ShapeDtypeStruct((nkey, d), jnp.float32),
        ],
        scratch_types=[
            pltpu.VMEM((rpw,), jnp.int32),
            pltpu.VMEM((nkey,), jnp.int32),
            pltpu.VMEM((cw, d), jnp.float32),
            pltpu.VMEM((cw, d), jnp.float32),
            pltpu.VMEM((kpw, d), jnp.float32),
            pltpu.SemaphoreType.DMA,
            pltpu.SemaphoreType.DMA,
            pltpu.SemaphoreType.DMA,
            pltpu.SemaphoreType.DMA,
        ],
    )
    def gather_kernel(table_hbm, keyn_hbm, gidx_hbm, kidx_hbm,
                      out1_hbm, out2_hbm,
                      gidx_v, kidx_v, buf0, buf1, krows_v,
                      sg0, sg1, sw, sk):
        wid = lax.axis_index("s") * 2 + lax.axis_index("c")
        base = wid * rpw
        bufs = (buf0, buf1)
        gsems = (sg0, sg1)

        pltpu.sync_copy(gidx_hbm.at[pl.ds(base, rpw)], gidx_v)

        pltpu.sync_copy(kidx_hbm, kidx_v)
        hk = pltpu.async_copy(
            keyn_hbm.at[kidx_v.at[pl.ds(wid * kpw, kpw)]], krows_v, sk)

        def g_start(c):
            return pltpu.async_copy(
                table_hbm.at[gidx_v.at[pl.ds(c * cw, cw)]],
                bufs[c % 2], gsems[c % 2])

        hg = {0: g_start(0)}
        if nchunk > 1:
            hg[1] = g_start(1)
        for c in range(nchunk):
            hg.pop(c).wait()
            hw = pltpu.async_copy(
                bufs[c % 2], out1_hbm.at[pl.ds(base + c * cw, cw)], sw)
            hw.wait()
            if c + 2 < nchunk:
                hg[c + 2] = g_start(c + 2)

        hk.wait()
        pltpu.sync_copy(krows_v, out2_hbm.at[pl.ds(wid * kpw, kpw)])

    return gather_kernel


def kernel(x_embed, prompt, prompt_key):
    b, s, d = x_embed.shape
    l, p, length, d2 = prompt.shape
    k = TOP_K

    x_mean = pl.pallas_call(
        _mean_body,
        grid=(b,),
        in_specs=[pl.BlockSpec((1, s, d), lambda i: (i, 0, 0))],
        out_specs=pl.BlockSpec((1, 1, d), lambda i: (i, 0, 0)),
        out_shape=jax.ShapeDtypeStruct((b, 1, d), jnp.float32),
    )(x_embed)
    x_mean = x_mean.reshape(b, d)

    sim, idx, key_norm, rs = pl.pallas_call(
        _simtopk_body,
        out_shape=[
            jax.ShapeDtypeStruct((b, p), jnp.float32),
            jax.ShapeDtypeStruct((b, k), jnp.int32),
            jax.ShapeDtypeStruct((p, d), jnp.float32),
            jax.ShapeDtypeStruct((1, 1), jnp.float32),
        ],
    )(x_mean, prompt_key)

    flat = idx.reshape(-1)
    table = jnp.transpose(prompt, (0, 2, 1, 3)).reshape(l * length * p, d)
    gidx = (idx[None, :, :, None]
            + (jnp.arange(l, dtype=jnp.int32) * length * p)[:, None, None, None]
            + (jnp.arange(length, dtype=jnp.int32) * p)[None, None, None, :]
            ).reshape(-1)
    out1, out2 = _sc_gather(l, p, length, d, b, k)(table, key_norm, gidx, flat)

    batched_prompt = out1.reshape(l, b, k * length, d)
    batched_key_norm = out2.reshape(b, k, d)
    reduce_sim = rs.reshape(())
    return (sim, idx, batched_prompt, batched_key_norm, reduce_sim)
